# baseline (speedup 1.0000x reference)
import numpy as np
import jax
import jax.numpy as jnp
from jax.experimental import pallas as pl
from jax.experimental.pallas import tpu as pltpu

_D = 2048
_ROWS = 1024
_NBUF = 3
_NEG = -1e30


def _np_consts():
    def verts(nbits):
        idx = np.arange(2 ** nbits)
        return (((idx[:, None] >> np.arange(nbits - 1, -1, -1)[None, :]) & 1) * 2.0 - 1.0).astype(np.float32)

    hexa = verts(6)
    q2v = verts(2)
    q3v = verts(3)

    q2g = np.zeros((4, 3), dtype=np.float32)
    for v, g in enumerate([2, 1, 1, 0]):
        q2g[v, g] = 1.0

    anch = np.stack([
        hexa[[63, 62]].mean(0),
        hexa[[19, 21]].mean(0),
        hexa[[0, 8]].mean(0),
    ]).astype(np.float32)
    anch = anch / np.clip(np.linalg.norm(anch, axis=-1, keepdims=True), 1e-12, None)

    bits = np.array([bin(i).count('1') for i in range(64)], dtype=np.float32)
    wa = (bits >= 5).astype(np.float32)
    wd = (bits == 3).astype(np.float32)
    wc = (bits <= 1).astype(np.float32)
    wht = np.stack([wa / wa.sum(), wd / wd.sum(), wc / wc.sum()], axis=1)

    return hexa, q2v, q3v, q2g, anch, wht


_HEXA, _Q2V, _Q3V, _Q2G, _ANCH, _WHT = _np_consts()

_VBASE = np.zeros((16, 384), dtype=np.float32)
_VBASE[0:2, 0:4] = _Q2V.T
_VBASE[2:5, 128:136] = _Q3V.T
_VBASE[5:11, 256:320] = _HEXA.T

_BIAS = np.full((1, 384), _NEG, dtype=np.float32)
_BIAS[0, 0:4] = 0.0
_BIAS[0, 128:136] = 0.0
_BIAS[0, 256:320] = 0.0

_G2PAD = np.zeros((384, 3), dtype=np.float32)
_G2PAD[0:4, :] = _Q2G
_S3PAD = np.zeros((384, 8), dtype=np.float32)
for _j in range(8):
    _S3PAD[128 + _j, _j] = 1.0
_G6COS = np.zeros((384, 3), dtype=np.float32)
_G6COS[256:320, :] = _HEXA @ _ANCH.T
_G6WHT = np.zeros((384, 3), dtype=np.float32)
_G6WHT[256:320, :] = _WHT


def _router_body(x_hbm, wp_ref, aux_ref, v_ref, bias_ref, gall_ref,
                 gw_ref, hex_ref, psum_ref, xbuf, sem):
    i = pl.program_id(0)
    nt = pl.num_programs(0)

    @pl.when(i == 0)
    def _():
        for k in range(_NBUF - 1):
            pltpu.make_async_copy(x_hbm.at[pl.ds(k * _ROWS, _ROWS), :],
                                  xbuf.at[k], sem.at[k]).start()

    nxt = i + _NBUF - 1

    @pl.when(nxt < nt)
    def _():
        slot = jax.lax.rem(nxt, _NBUF)
        pltpu.make_async_copy(x_hbm.at[pl.ds(nxt * _ROWS, _ROWS), :],
                              xbuf.at[slot], sem.at[slot]).start()

    slot = jax.lax.rem(i, _NBUF)
    pltpu.make_async_copy(xbuf.at[slot], xbuf.at[slot], sem.at[slot]).wait()
    x = xbuf[slot]
    t = jnp.dot(x, wp_ref[...], preferred_element_type=jnp.float32)
    s1 = jnp.sum(x, axis=1, keepdims=True)
    s2 = jnp.sum(x * x, axis=1, keepdims=True)
    mu = s1 * (1.0 / _D)
    var = s2 * (1.0 / _D) - mu * mu
    rstd = jax.lax.rsqrt(var + 1e-5)
    cs = aux_ref[0:1, :]
    bp = aux_ref[1:2, :]
    p = jnp.tanh((t - mu * cs) * rstd + bp)

    logits = jnp.dot(p, v_ref[...], preferred_element_type=jnp.float32) + bias_ref[...]

    def seg_softmax(l):
        m = jnp.max(l, axis=1, keepdims=True)
        e = jnp.exp(l - m)
        return e * (1.0 / jnp.sum(e, axis=1, keepdims=True))

    w2 = seg_softmax(logits[:, 0:128])
    w3 = seg_softmax(logits[:, 128:256])
    w6 = seg_softmax(logits[:, 256:384])
    hex_ref[...] = w6[:, 0:64]

    wall = jnp.concatenate([w2, w3, w6], axis=1)
    gl = jnp.dot(wall, gall_ref[...], preferred_element_type=jnp.float32)
    gm = jnp.max(gl, axis=1, keepdims=True)
    ge = jnp.exp(gl - gm)
    gw = ge * (1.0 / jnp.sum(ge, axis=1, keepdims=True))
    gw_ref[...] = gw

    hs = jnp.sum(w6, axis=0, keepdims=True)
    gs = jnp.sum(gw, axis=0, keepdims=True)
    gs_pad = jnp.concatenate([gs, jnp.zeros((1, 125), jnp.float32)], axis=1)
    psum_ref[...] = jnp.concatenate([hs, gs_pad], axis=0).reshape(1, 2, 128)


def kernel(x, ln_w, ln_b, w_q2, w_q3, w_q6, log_temp, log_scale_mix, q3_to_group, log_wht_mix):
    B, T, D = x.shape
    n = B * T
    ntiles = n // _ROWS
    x2 = x.reshape(n, D)

    temp = jnp.clip(jnp.exp(log_temp), 0.1, 5.0)
    mix = jax.nn.softmax(log_scale_mix)
    alpha = jax.nn.sigmoid(log_wht_mix)

    w = jnp.concatenate([w_q2, w_q3, w_q6], axis=0)
    wp = jnp.concatenate([(w * ln_w[None, :]).T,
                          jnp.zeros((D, 5), jnp.float32)], axis=1)
    cs = jnp.sum(wp, axis=0, keepdims=True)
    bp = jnp.concatenate([(w @ ln_b)[None, :], jnp.zeros((1, 5), jnp.float32)], axis=1)
    aux = jnp.concatenate([cs, bp], axis=0)

    vmat = jnp.asarray(_VBASE) * (1.0 / temp)
    bias = jnp.asarray(_BIAS)

    sm3 = jax.nn.softmax(q3_to_group, axis=-1)
    gall = (mix[0] * jnp.asarray(_G2PAD)
            + mix[1] * (jnp.asarray(_S3PAD) @ sm3)
            + mix[2] * ((1.0 - alpha) * jnp.asarray(_G6COS)
                        + alpha * jnp.asarray(_G6WHT)))

    gw2, hexw2, psum = pl.pallas_call(
        _router_body,
        grid=(ntiles,),
        in_specs=[
            pl.BlockSpec(memory_space=pl.ANY),
            pl.BlockSpec((D, 16), lambda i: (0, 0)),
            pl.BlockSpec((2, 16), lambda i: (0, 0)),
            pl.BlockSpec((16, 384), lambda i: (0, 0)),
            pl.BlockSpec((1, 384), lambda i: (0, 0)),
            pl.BlockSpec((384, 3), lambda i: (0, 0)),
        ],
        out_specs=[
            pl.BlockSpec((_ROWS, 3), lambda i: (i, 0)),
            pl.BlockSpec((_ROWS, 64), lambda i: (i, 0)),
            pl.BlockSpec((1, 2, 128), lambda i: (i, 0, 0)),
        ],
        out_shape=[
            jax.ShapeDtypeStruct((n, 3), jnp.float32),
            jax.ShapeDtypeStruct((n, 64), jnp.float32),
            jax.ShapeDtypeStruct((ntiles, 2, 128), jnp.float32),
        ],
        scratch_shapes=[
            pltpu.VMEM((_NBUF, _ROWS, D), jnp.float32),
            pltpu.SemaphoreType.DMA((_NBUF,)),
        ],
        compiler_params=pltpu.CompilerParams(
            dimension_semantics=("arbitrary",),
        ),
        name="msg_router",
    )(x2, wp, aux, vmat, bias, gall)

    ps = jnp.sum(psum, axis=0)
    mh = ps[0, 0:64] * (1.0 / n)
    mg = ps[1, 0:3] * (1.0 / n)
    lb_loss = jnp.sum(mg * jnp.log(mg + 1e-8)) + 0.1 * jnp.sum(mh * jnp.log(mh + 1e-8))

    return gw2.reshape(B, T, 3), hexw2.reshape(B, T, 64), lb_loss


# speedup vs baseline: 1.0281x; 1.0281x over previous
import numpy as np
import jax
import jax.numpy as jnp
from jax.experimental import pallas as pl
from jax.experimental.pallas import tpu as pltpu

_D = 2048
_ROWS = 2048
_NBUF = 3
_NEG = -1e30


def _np_consts():
    def verts(nbits):
        idx = np.arange(2 ** nbits)
        return (((idx[:, None] >> np.arange(nbits - 1, -1, -1)[None, :]) & 1) * 2.0 - 1.0).astype(np.float32)

    hexa = verts(6)
    q2v = verts(2)
    q3v = verts(3)

    q2g = np.zeros((4, 3), dtype=np.float32)
    for v, g in enumerate([2, 1, 1, 0]):
        q2g[v, g] = 1.0

    anch = np.stack([
        hexa[[63, 62]].mean(0),
        hexa[[19, 21]].mean(0),
        hexa[[0, 8]].mean(0),
    ]).astype(np.float32)
    anch = anch / np.clip(np.linalg.norm(anch, axis=-1, keepdims=True), 1e-12, None)

    bits = np.array([bin(i).count('1') for i in range(64)], dtype=np.float32)
    wa = (bits >= 5).astype(np.float32)
    wd = (bits == 3).astype(np.float32)
    wc = (bits <= 1).astype(np.float32)
    wht = np.stack([wa / wa.sum(), wd / wd.sum(), wc / wc.sum()], axis=1)

    return hexa, q2v, q3v, q2g, anch, wht


_HEXA, _Q2V, _Q3V, _Q2G, _ANCH, _WHT = _np_consts()

_VBASE = np.zeros((16, 384), dtype=np.float32)
_VBASE[0:2, 0:4] = _Q2V.T
_VBASE[2:5, 128:136] = _Q3V.T
_VBASE[5:11, 256:320] = _HEXA.T

_BIAS = np.full((1, 384), _NEG, dtype=np.float32)
_BIAS[0, 0:4] = 0.0
_BIAS[0, 128:136] = 0.0
_BIAS[0, 256:320] = 0.0

_G2PAD = np.zeros((384, 3), dtype=np.float32)
_G2PAD[0:4, :] = _Q2G
_S3PAD = np.zeros((384, 8), dtype=np.float32)
for _j in range(8):
    _S3PAD[128 + _j, _j] = 1.0
_G6COS = np.zeros((384, 3), dtype=np.float32)
_G6COS[256:320, :] = _HEXA @ _ANCH.T
_G6WHT = np.zeros((384, 3), dtype=np.float32)
_G6WHT[256:320, :] = _WHT


def _router_body(x_hbm, wp_ref, aux_ref, v_ref, bias_ref, gall_ref,
                 gw_ref, hex_ref, psum_ref, xbuf, sem):
    i = pl.program_id(0)
    nt = pl.num_programs(0)

    @pl.when(i == 0)
    def _():
        for k in range(_NBUF - 1):
            pltpu.make_async_copy(x_hbm.at[pl.ds(k * _ROWS, _ROWS), :],
                                  xbuf.at[k], sem.at[k]).start()

    nxt = i + _NBUF - 1

    @pl.when(nxt < nt)
    def _():
        slot = jax.lax.rem(nxt, _NBUF)
        pltpu.make_async_copy(x_hbm.at[pl.ds(nxt * _ROWS, _ROWS), :],
                              xbuf.at[slot], sem.at[slot]).start()

    slot = jax.lax.rem(i, _NBUF)
    pltpu.make_async_copy(xbuf.at[slot], xbuf.at[slot], sem.at[slot]).wait()
    x = xbuf[slot]
    t = jnp.dot(x, wp_ref[...], preferred_element_type=jnp.float32)
    s1 = jnp.sum(x, axis=1, keepdims=True)
    s2 = jnp.sum(x * x, axis=1, keepdims=True)
    mu = s1 * (1.0 / _D)
    var = s2 * (1.0 / _D) - mu * mu
    rstd = jax.lax.rsqrt(var + 1e-5)
    cs = aux_ref[0:1, :]
    bp = aux_ref[1:2, :]
    p = jnp.tanh((t - mu * cs) * rstd + bp)

    logits = jnp.dot(p, v_ref[...], preferred_element_type=jnp.float32) + bias_ref[...]

    def seg_softmax(l):
        m = jnp.max(l, axis=1, keepdims=True)
        e = jnp.exp(l - m)
        return e * (1.0 / jnp.sum(e, axis=1, keepdims=True))

    w2 = seg_softmax(logits[:, 0:128])
    w3 = seg_softmax(logits[:, 128:256])
    w6 = seg_softmax(logits[:, 256:384])
    hex_ref[...] = w6[:, 0:64]

    wall = jnp.concatenate([w2, w3, w6], axis=1)
    gl = jnp.dot(wall, gall_ref[...], preferred_element_type=jnp.float32)
    gm = jnp.max(gl, axis=1, keepdims=True)
    ge = jnp.exp(gl - gm)
    gw = ge * (1.0 / jnp.sum(ge, axis=1, keepdims=True))
    gw_ref[...] = gw

    hs = jnp.sum(w6, axis=0, keepdims=True)
    gs = jnp.sum(gw, axis=0, keepdims=True)
    gs_pad = jnp.concatenate([gs, jnp.zeros((1, 125), jnp.float32)], axis=1)
    psum_ref[...] = jnp.concatenate([hs, gs_pad], axis=0).reshape(1, 2, 128)


def kernel(x, ln_w, ln_b, w_q2, w_q3, w_q6, log_temp, log_scale_mix, q3_to_group, log_wht_mix):
    B, T, D = x.shape
    n = B * T
    ntiles = n // _ROWS
    x2 = x.reshape(n, D)

    temp = jnp.clip(jnp.exp(log_temp), 0.1, 5.0)
    mix = jax.nn.softmax(log_scale_mix)
    alpha = jax.nn.sigmoid(log_wht_mix)

    w = jnp.concatenate([w_q2, w_q3, w_q6], axis=0)
    wp = jnp.concatenate([(w * ln_w[None, :]).T,
                          jnp.zeros((D, 5), jnp.float32)], axis=1)
    cs = jnp.sum(wp, axis=0, keepdims=True)
    bp = jnp.concatenate([(w @ ln_b)[None, :], jnp.zeros((1, 5), jnp.float32)], axis=1)
    aux = jnp.concatenate([cs, bp], axis=0)

    vmat = jnp.asarray(_VBASE) * (1.0 / temp)
    bias = jnp.asarray(_BIAS)

    sm3 = jax.nn.softmax(q3_to_group, axis=-1)
    gall = (mix[0] * jnp.asarray(_G2PAD)
            + mix[1] * (jnp.asarray(_S3PAD) @ sm3)
            + mix[2] * ((1.0 - alpha) * jnp.asarray(_G6COS)
                        + alpha * jnp.asarray(_G6WHT)))

    gw2, hexw2, psum = pl.pallas_call(
        _router_body,
        grid=(ntiles,),
        in_specs=[
            pl.BlockSpec(memory_space=pl.ANY),
            pl.BlockSpec((D, 16), lambda i: (0, 0)),
            pl.BlockSpec((2, 16), lambda i: (0, 0)),
            pl.BlockSpec((16, 384), lambda i: (0, 0)),
            pl.BlockSpec((1, 384), lambda i: (0, 0)),
            pl.BlockSpec((384, 3), lambda i: (0, 0)),
        ],
        out_specs=[
            pl.BlockSpec((_ROWS, 3), lambda i: (i, 0)),
            pl.BlockSpec((_ROWS, 64), lambda i: (i, 0)),
            pl.BlockSpec((1, 2, 128), lambda i: (i, 0, 0)),
        ],
        out_shape=[
            jax.ShapeDtypeStruct((n, 3), jnp.float32),
            jax.ShapeDtypeStruct((n, 64), jnp.float32),
            jax.ShapeDtypeStruct((ntiles, 2, 128), jnp.float32),
        ],
        scratch_shapes=[
            pltpu.VMEM((_NBUF, _ROWS, D), jnp.float32),
            pltpu.SemaphoreType.DMA((_NBUF,)),
        ],
        compiler_params=pltpu.CompilerParams(
            dimension_semantics=("arbitrary",),
        ),
        name="msg_router",
    )(x2, wp, aux, vmat, bias, gall)

    ps = jnp.sum(psum, axis=0)
    mh = ps[0, 0:64] * (1.0 / n)
    mg = ps[1, 0:3] * (1.0 / n)
    lb_loss = jnp.sum(mg * jnp.log(mg + 1e-8)) + 0.1 * jnp.sum(mh * jnp.log(mh + 1e-8))

    return gw2.reshape(B, T, 3), hexw2.reshape(B, T, 64), lb_loss


# speedup vs baseline: 1.0993x; 1.0692x over previous
import numpy as np
import jax
import jax.numpy as jnp
from jax.experimental import pallas as pl
from jax.experimental.pallas import tpu as pltpu

_D = 2048
_ROWS = 2048
_NEG = -1e30


def _np_consts():
    def verts(nbits):
        idx = np.arange(2 ** nbits)
        return (((idx[:, None] >> np.arange(nbits - 1, -1, -1)[None, :]) & 1) * 2.0 - 1.0).astype(np.float32)

    hexa = verts(6)
    q2v = verts(2)
    q3v = verts(3)

    q2g = np.zeros((4, 3), dtype=np.float32)
    for v, g in enumerate([2, 1, 1, 0]):
        q2g[v, g] = 1.0

    anch = np.stack([
        hexa[[63, 62]].mean(0),
        hexa[[19, 21]].mean(0),
        hexa[[0, 8]].mean(0),
    ]).astype(np.float32)
    anch = anch / np.clip(np.linalg.norm(anch, axis=-1, keepdims=True), 1e-12, None)

    bits = np.array([bin(i).count('1') for i in range(64)], dtype=np.float32)
    wa = (bits >= 5).astype(np.float32)
    wd = (bits == 3).astype(np.float32)
    wc = (bits <= 1).astype(np.float32)
    wht = np.stack([wa / wa.sum(), wd / wd.sum(), wc / wc.sum()], axis=1)

    return hexa, q2v, q3v, q2g, anch, wht


_HEXA, _Q2V, _Q3V, _Q2G, _ANCH, _WHT = _np_consts()

_VBASE = np.zeros((16, 384), dtype=np.float32)
_VBASE[0:2, 0:4] = _Q2V.T
_VBASE[2:5, 128:136] = _Q3V.T
_VBASE[5:11, 256:320] = _HEXA.T

_BIAS = np.full((1, 384), _NEG, dtype=np.float32)
_BIAS[0, 0:4] = 0.0
_BIAS[0, 128:136] = 0.0
_BIAS[0, 256:320] = 0.0

_G2PAD = np.zeros((384, 3), dtype=np.float32)
_G2PAD[0:4, :] = _Q2G
_S3PAD = np.zeros((384, 8), dtype=np.float32)
for _j in range(8):
    _S3PAD[128 + _j, _j] = 1.0
_G6COS = np.zeros((384, 3), dtype=np.float32)
_G6COS[256:320, :] = _HEXA @ _ANCH.T
_G6WHT = np.zeros((384, 3), dtype=np.float32)
_G6WHT[256:320, :] = _WHT


def _router_body(x_ref, wp_ref, aux_ref, v_ref, bias_ref, gall_ref,
                 gw_ref, hex_ref, psum_ref):
    x = x_ref[...]
    t = jnp.dot(x, wp_ref[...], preferred_element_type=jnp.float32)
    s1 = jnp.sum(x, axis=1, keepdims=True)
    s2 = jnp.sum(x * x, axis=1, keepdims=True)
    mu = s1 * (1.0 / _D)
    var = s2 * (1.0 / _D) - mu * mu
    rstd = jax.lax.rsqrt(var + 1e-5)
    cs = aux_ref[0:1, :]
    bp = aux_ref[1:2, :]
    p = jnp.tanh((t - mu * cs) * rstd + bp)

    logits = jnp.dot(p, v_ref[...], preferred_element_type=jnp.float32) + bias_ref[...]

    def seg_softmax(l):
        m = jnp.max(l, axis=1, keepdims=True)
        e = jnp.exp(l - m)
        return e * (1.0 / jnp.sum(e, axis=1, keepdims=True))

    w2 = seg_softmax(logits[:, 0:128])
    w3 = seg_softmax(logits[:, 128:256])
    w6 = seg_softmax(logits[:, 256:384])
    hex_ref[...] = w6[:, 0:64]

    wall = jnp.concatenate([w2, w3, w6], axis=1)
    gl = jnp.dot(wall, gall_ref[...], preferred_element_type=jnp.float32)
    gm = jnp.max(gl, axis=1, keepdims=True)
    ge = jnp.exp(gl - gm)
    gw = ge * (1.0 / jnp.sum(ge, axis=1, keepdims=True))
    gw_ref[...] = gw

    hs = jnp.sum(w6, axis=0, keepdims=True)
    gs = jnp.sum(gw, axis=0, keepdims=True)
    gs_pad = jnp.concatenate([gs, jnp.zeros((1, 125), jnp.float32)], axis=1)
    psum_ref[...] = jnp.concatenate([hs, gs_pad], axis=0).reshape(1, 2, 128)


def kernel(x, ln_w, ln_b, w_q2, w_q3, w_q6, log_temp, log_scale_mix, q3_to_group, log_wht_mix):
    B, T, D = x.shape
    n = B * T
    ntiles = n // _ROWS
    x2 = x.reshape(n, D)

    temp = jnp.clip(jnp.exp(log_temp), 0.1, 5.0)
    mix = jax.nn.softmax(log_scale_mix)
    alpha = jax.nn.sigmoid(log_wht_mix)

    w = jnp.concatenate([w_q2, w_q3, w_q6], axis=0)
    wp = jnp.concatenate([(w * ln_w[None, :]).T,
                          jnp.zeros((D, 5), jnp.float32)], axis=1)
    cs = jnp.sum(wp, axis=0, keepdims=True)
    bp = jnp.concatenate([(w @ ln_b)[None, :], jnp.zeros((1, 5), jnp.float32)], axis=1)
    aux = jnp.concatenate([cs, bp], axis=0)

    vmat = jnp.asarray(_VBASE) * (1.0 / temp)
    bias = jnp.asarray(_BIAS)

    sm3 = jax.nn.softmax(q3_to_group, axis=-1)
    gall = (mix[0] * jnp.asarray(_G2PAD)
            + mix[1] * (jnp.asarray(_S3PAD) @ sm3)
            + mix[2] * ((1.0 - alpha) * jnp.asarray(_G6COS)
                        + alpha * jnp.asarray(_G6WHT)))

    gw2, hexw2, psum = pl.pallas_call(
        _router_body,
        grid=(ntiles,),
        in_specs=[
            pl.BlockSpec((_ROWS, D), lambda i: (i, 0)),
            pl.BlockSpec((D, 16), lambda i: (0, 0)),
            pl.BlockSpec((2, 16), lambda i: (0, 0)),
            pl.BlockSpec((16, 384), lambda i: (0, 0)),
            pl.BlockSpec((1, 384), lambda i: (0, 0)),
            pl.BlockSpec((384, 3), lambda i: (0, 0)),
        ],
        out_specs=[
            pl.BlockSpec((_ROWS, 3), lambda i: (i, 0)),
            pl.BlockSpec((_ROWS, 64), lambda i: (i, 0)),
            pl.BlockSpec((1, 2, 128), lambda i: (i, 0, 0)),
        ],
        out_shape=[
            jax.ShapeDtypeStruct((n, 3), jnp.float32),
            jax.ShapeDtypeStruct((n, 64), jnp.float32),
            jax.ShapeDtypeStruct((ntiles, 2, 128), jnp.float32),
        ],
        compiler_params=pltpu.CompilerParams(
            dimension_semantics=("parallel",),
        ),
        name="msg_router",
    )(x2, wp, aux, vmat, bias, gall)

    ps = jnp.sum(psum, axis=0)
    mh = ps[0, 0:64] * (1.0 / n)
    mg = ps[1, 0:3] * (1.0 / n)
    lb_loss = jnp.sum(mg * jnp.log(mg + 1e-8)) + 0.1 * jnp.sum(mh * jnp.log(mh + 1e-8))

    return gw2.reshape(B, T, 3), hexw2.reshape(B, T, 64), lb_loss
